# baseline (device time: 265880 ns/iter reference)
import jax
import jax.numpy as jnp
from jax import lax
from jax.experimental import pallas as pl
from jax.experimental.pallas import tpu as pltpu


def kernel(Q, K, V):
    b, sq, h, d = Q.shape
    skv = K.shape[1]
    skv2 = skv // 2
    BB = 1
    scale = d ** -0.5

    xs = jnp.reshape(lax.axis_index("x"), (1,)).astype(jnp.int32)

    def body(x_sref, q_ref, k_ref, v_ref, o_ref,
             acc_ref, l_ref, racc_ref, rl_ref, racc2_ref, rl2_ref,
             send_sems, recv_sems):
        bi = pl.program_id(0)
        nb = pl.num_programs(0)
        my_x = lax.axis_index("x")
        my_y = lax.axis_index("y")
        x_nbr = (1 - my_x, my_y)
        y_nbr = (my_x, 1 - my_y)

        @pl.when(bi == 0)
        def _():
            barrier_sem = pltpu.get_barrier_semaphore()
            for nbr in (x_nbr, y_nbr):
                pl.semaphore_signal(
                    barrier_sem, inc=1,
                    device_id=nbr, device_id_type=pl.DeviceIdType.MESH,
                )
            pl.semaphore_wait(barrier_sem, 2)

        q4 = q_ref[:, 0] * scale
        k4 = k_ref[...]
        v4 = v_ref[...]

        srow = jnp.sum(k4 * q4[:, None], axis=3, keepdims=True)
        pfull = jnp.exp(jnp.broadcast_to(srow, k4.shape))
        accb = jnp.sum(v4 * pfull, axis=1)
        lfull = jnp.sum(pfull, axis=1)

        acc_ref[pl.ds(bi * BB, BB)] = accb
        l_ref[pl.ds(bi * BB, BB)] = lfull

        @pl.when(bi == nb - 1)
        def _():
            rdma_acc = pltpu.make_async_remote_copy(
                src_ref=acc_ref, dst_ref=racc_ref,
                send_sem=send_sems.at[0], recv_sem=recv_sems.at[0],
                device_id=x_nbr, device_id_type=pl.DeviceIdType.MESH,
            )
            rdma_l = pltpu.make_async_remote_copy(
                src_ref=l_ref, dst_ref=rl_ref,
                send_sem=send_sems.at[1], recv_sem=recv_sems.at[1],
                device_id=x_nbr, device_id_type=pl.DeviceIdType.MESH,
            )
            rdma_acc.start()
            rdma_l.start()
            rdma_acc.wait()
            rdma_l.wait()
            acc_ref[...] = acc_ref[...] + racc_ref[...]
            l_ref[...] = l_ref[...] + rl_ref[...]

            rdma_acc2 = pltpu.make_async_remote_copy(
                src_ref=acc_ref, dst_ref=racc2_ref,
                send_sem=send_sems.at[2], recv_sem=recv_sems.at[2],
                device_id=y_nbr, device_id_type=pl.DeviceIdType.MESH,
            )
            rdma_l2 = pltpu.make_async_remote_copy(
                src_ref=l_ref, dst_ref=rl2_ref,
                send_sem=send_sems.at[3], recv_sem=recv_sems.at[3],
                device_id=y_nbr, device_id_type=pl.DeviceIdType.MESH,
            )
            rdma_acc2.start()
            rdma_l2.start()
            rdma_acc2.wait()
            rdma_l2.wait()

            lsum = l_ref[...] + rl2_ref[...]
            accs = acc_ref[...] + racc2_ref[...]
            o_ref[...] = (accs / lsum)[:, None, :, :]

    grid_spec = pltpu.PrefetchScalarGridSpec(
        num_scalar_prefetch=1,
        grid=(b // BB,),
        in_specs=[
            pl.BlockSpec((BB, 1, h, d), lambda i, xs: (i, 0, 0, 0)),
            pl.BlockSpec((BB, skv2, h, d), lambda i, xs: (i, xs[0], 0, 0)),
            pl.BlockSpec((BB, skv2, h, d), lambda i, xs: (i, xs[0], 0, 0)),
        ],
        out_specs=pl.BlockSpec((b, sq, h, d), lambda i, xs: (0, 0, 0, 0)),
        scratch_shapes=[
            pltpu.VMEM((b, h, d), jnp.float32),
            pltpu.VMEM((b, h, d), jnp.float32),
            pltpu.VMEM((b, h, d), jnp.float32),
            pltpu.VMEM((b, h, d), jnp.float32),
            pltpu.VMEM((b, h, d), jnp.float32),
            pltpu.VMEM((b, h, d), jnp.float32),
            pltpu.SemaphoreType.DMA((4,)),
            pltpu.SemaphoreType.DMA((4,)),
        ],
    )
    return pl.pallas_call(
        body,
        grid_spec=grid_spec,
        out_shape=jax.ShapeDtypeStruct((b, sq, h, d), jnp.float32),
        compiler_params=pltpu.CompilerParams(collective_id=0),
    )(xs, Q, K, V)


# device time: 160975 ns/iter; 1.6517x vs baseline; 1.6517x over previous
import jax
import jax.numpy as jnp
from jax import lax
from jax.experimental import pallas as pl
from jax.experimental.pallas import tpu as pltpu


def kernel(Q, K, V):
    b, sq, h, d = Q.shape
    skv = K.shape[1]
    skv2 = skv // 2
    hd = h * d
    scale = d ** -0.5

    Qt = Q.reshape(b, hd, 1)
    my_x = lax.axis_index("x")
    Kh = lax.dynamic_slice_in_dim(K, my_x * skv2, skv2, axis=1)
    Vh = lax.dynamic_slice_in_dim(V, my_x * skv2, skv2, axis=1)
    Kf = Kh.reshape(b, skv2, hd)
    Vf = Vh.reshape(b, skv2, hd)
    xs = jnp.reshape(my_x, (1,)).astype(jnp.int32)

    def body(x_sref, q_ref, k_ref, v_ref, o_ref,
             acc_ref, l_ref, racc_ref, rl_ref, racc2_ref, rl2_ref,
             send_sems, recv_sems):
        bi = pl.program_id(0)
        nb = pl.num_programs(0)
        my_x = lax.axis_index("x")
        my_y = lax.axis_index("y")
        x_nbr = (1 - my_x, my_y)
        y_nbr = (my_x, 1 - my_y)

        @pl.when(bi == 0)
        def _():
            barrier_sem = pltpu.get_barrier_semaphore()
            for nbr in (x_nbr, y_nbr):
                pl.semaphore_signal(
                    barrier_sem, inc=1,
                    device_id=nbr, device_id_type=pl.DeviceIdType.MESH,
                )
            pl.semaphore_wait(barrier_sem, 2)

        rows = lax.broadcasted_iota(jnp.int32, (h, hd), 0)
        cols = lax.broadcasted_iota(jnp.int32, (h, hd), 1)
        maskf = ((cols // d) == rows).astype(jnp.float32)
        rowsT = lax.broadcasted_iota(jnp.int32, (hd, h), 0)
        colsT = lax.broadcasted_iota(jnp.int32, (hd, h), 1)
        maskT = (rowsT // d) == colsT

        qcol = q_ref[0] * scale
        qdT = jnp.where(maskT, qcol, 0.0)
        k2 = k_ref[0]
        v2 = v_ref[0]

        st = lax.dot_general(k2, qdT, (((1,), (0,)), ((), ())),
                             preferred_element_type=jnp.float32)
        pt = jnp.exp(st)
        lrow = lax.dot_general(jnp.ones((1, skv2), jnp.float32), pt,
                               (((1,), (0,)), ((), ())),
                               preferred_element_type=jnp.float32)
        r = lax.dot_general(pt, v2, (((0,), (0,)), ((), ())),
                            preferred_element_type=jnp.float32)
        accrow = jnp.sum(r * maskf, axis=0, keepdims=True)

        acc_ref[pl.ds(bi, 1), :] = accrow
        l_ref[pl.ds(bi, 1), :] = lrow

        @pl.when(bi == nb - 1)
        def _():
            rdma_acc = pltpu.make_async_remote_copy(
                src_ref=acc_ref, dst_ref=racc_ref,
                send_sem=send_sems.at[0], recv_sem=recv_sems.at[0],
                device_id=x_nbr, device_id_type=pl.DeviceIdType.MESH,
            )
            rdma_l = pltpu.make_async_remote_copy(
                src_ref=l_ref, dst_ref=rl_ref,
                send_sem=send_sems.at[1], recv_sem=recv_sems.at[1],
                device_id=x_nbr, device_id_type=pl.DeviceIdType.MESH,
            )
            rdma_acc.start()
            rdma_l.start()
            rdma_acc.wait()
            rdma_l.wait()
            acc_ref[...] = acc_ref[...] + racc_ref[...]
            l_ref[...] = l_ref[...] + rl_ref[...]

            rdma_acc2 = pltpu.make_async_remote_copy(
                src_ref=acc_ref, dst_ref=racc2_ref,
                send_sem=send_sems.at[2], recv_sem=recv_sems.at[2],
                device_id=y_nbr, device_id_type=pl.DeviceIdType.MESH,
            )
            rdma_l2 = pltpu.make_async_remote_copy(
                src_ref=l_ref, dst_ref=rl2_ref,
                send_sem=send_sems.at[3], recv_sem=recv_sems.at[3],
                device_id=y_nbr, device_id_type=pl.DeviceIdType.MESH,
            )
            rdma_acc2.start()
            rdma_l2.start()
            rdma_acc2.wait()
            rdma_l2.wait()

            lsum = l_ref[...] + rl2_ref[...]
            lfull = lax.dot_general(lsum, maskf, (((1,), (0,)), ((), ())),
                                    preferred_element_type=jnp.float32)
            o_ref[...] = (acc_ref[...] + racc2_ref[...]) / lfull

    grid_spec = pltpu.PrefetchScalarGridSpec(
        num_scalar_prefetch=1,
        grid=(b,),
        in_specs=[
            pl.BlockSpec((1, hd, 1), lambda i, xs: (i, 0, 0)),
            pl.BlockSpec((1, skv2, hd), lambda i, xs: (i, 0, 0)),
            pl.BlockSpec((1, skv2, hd), lambda i, xs: (i, 0, 0)),
        ],
        out_specs=pl.BlockSpec((b, hd), lambda i, xs: (0, 0)),
        scratch_shapes=[
            pltpu.VMEM((b, hd), jnp.float32),
            pltpu.VMEM((b, h), jnp.float32),
            pltpu.VMEM((b, hd), jnp.float32),
            pltpu.VMEM((b, h), jnp.float32),
            pltpu.VMEM((b, hd), jnp.float32),
            pltpu.VMEM((b, h), jnp.float32),
            pltpu.SemaphoreType.DMA((4,)),
            pltpu.SemaphoreType.DMA((4,)),
        ],
    )
    out = pl.pallas_call(
        body,
        grid_spec=grid_spec,
        out_shape=jax.ShapeDtypeStruct((b, hd), jnp.float32),
        compiler_params=pltpu.CompilerParams(collective_id=0),
    )(xs, Qt, Kf, Vf)
    return out.reshape(b, sq, h, d)
